# Initial kernel scaffold; baseline (speedup 1.0000x reference)
import jax
import jax.numpy as jnp
from jax.experimental import pallas as pl


def kernel(indices, w_es, instance, tf_table, idf_table):
    raise NotImplementedError("write your pallas kernel here")


# speedup vs baseline: 1.0579x; 1.0579x over previous
import functools

import jax
import jax.numpy as jnp
from jax import lax
from jax.experimental import pallas as pl
from jax.experimental.pallas import tpu as pltpu
from jax.experimental.pallas import tpu_sc as plsc

B, L, V = 4096, 200, 1000000
N = B * L
NC, NS, LANES = 2, 16, 16
NW = NC * NS
PER_W = N // NW


def _tfidf_body(idx_hbm, tf_hbm, idf_hbm, out_hbm,
                idx_v, tf_v, idf_v, sem_tf, sem_idf):
    wid = lax.axis_index("s") * NC + lax.axis_index("c")
    base = wid * PER_W

    pltpu.sync_copy(idx_hbm.at[pl.ds(base, PER_W)], idx_v)
    cp_tf = pltpu.async_copy(tf_hbm.at[idx_v], tf_v, sem_tf)
    cp_idf = pltpu.async_copy(idf_hbm.at[idx_v], idf_v, sem_idf)
    cp_tf.wait()
    cp_idf.wait()

    def mul_step(i, _):
        sl = pl.ds(i * LANES, LANES)
        tf_v[sl] = tf_v[sl] * idf_v[sl]
        return 0

    lax.fori_loop(0, PER_W // LANES, mul_step, 0, unroll=4)
    pltpu.sync_copy(tf_v, out_hbm.at[pl.ds(base, PER_W)])


@jax.jit
def _tfidf_sc(idx_flat, tf_flat, idf_flat):
    mesh = plsc.VectorSubcoreMesh(core_axis_name="c", subcore_axis_name="s")
    fn = pl.kernel(
        _tfidf_body,
        out_type=jax.ShapeDtypeStruct((N,), jnp.float32),
        mesh=mesh,
        scratch_types=[
            pltpu.VMEM((PER_W,), jnp.int32),
            pltpu.VMEM((PER_W,), jnp.float32),
            pltpu.VMEM((PER_W,), jnp.float32),
            pltpu.SemaphoreType.DMA,
            pltpu.SemaphoreType.DMA,
        ],
    )
    return fn(idx_flat, tf_flat, idf_flat)


def kernel(indices, w_es, instance, tf_table, idf_table):
    idx_flat = indices.reshape(N).astype(jnp.int32)
    tf_flat = tf_table.reshape(V)
    idf_flat = idf_table.reshape(V)
    out = _tfidf_sc(idx_flat, tf_flat, idf_flat)
    return out.reshape(B, L)


# speedup vs baseline: 1.1621x; 1.0986x over previous
import functools

import jax
import jax.numpy as jnp
from jax import lax
from jax.experimental import pallas as pl
from jax.experimental.pallas import tpu as pltpu
from jax.experimental.pallas import tpu_sc as plsc

B, L, V = 4096, 200, 1000000
N = B * L
NC, NS, LANES = 2, 16, 16
NW = NC * NS
PER_W = N // NW
PR, PC = 500, 2000


def _prod_body(tf_ref, idf_ref, out_ref):
    out_ref[...] = tf_ref[...] * idf_ref[...]


def _gather_body(idx_hbm, comb_hbm, out_hbm, idx_v, val_v, sem):
    wid = lax.axis_index("s") * NC + lax.axis_index("c")
    base = wid * PER_W
    pltpu.sync_copy(idx_hbm.at[pl.ds(base, PER_W)], idx_v)
    pltpu.async_copy(comb_hbm.at[idx_v], val_v, sem).wait()
    pltpu.sync_copy(val_v, out_hbm.at[pl.ds(base, PER_W)])


@jax.jit
def _tfidf(idx_flat, tf_flat, idf_flat):
    combined = pl.pallas_call(
        _prod_body,
        out_shape=jax.ShapeDtypeStruct((PR, PC), jnp.float32),
    )(tf_flat.reshape(PR, PC), idf_flat.reshape(PR, PC))

    mesh = plsc.VectorSubcoreMesh(core_axis_name="c", subcore_axis_name="s")
    fn = pl.kernel(
        _gather_body,
        out_type=jax.ShapeDtypeStruct((N,), jnp.float32),
        mesh=mesh,
        scratch_types=[
            pltpu.VMEM((PER_W,), jnp.int32),
            pltpu.VMEM((PER_W,), jnp.float32),
            pltpu.SemaphoreType.DMA,
        ],
    )
    return fn(idx_flat, combined.reshape(V))


def kernel(indices, w_es, instance, tf_table, idf_table):
    idx_flat = indices.reshape(N).astype(jnp.int32)
    tf_flat = tf_table.reshape(V)
    idf_flat = idf_table.reshape(V)
    out = _tfidf(idx_flat, tf_flat, idf_flat)
    return out.reshape(B, L)


# speedup vs baseline: 1.3143x; 1.1309x over previous
import functools

import jax
import jax.numpy as jnp
from jax import lax
from jax.experimental import pallas as pl
from jax.experimental.pallas import tpu as pltpu
from jax.experimental.pallas import tpu_sc as plsc

B, L, V = 4096, 200, 1000000
N = B * L
NC, NS, LANES = 2, 16, 16
NW = NC * NS
PER_W = N // NW

PROD_PER_S = 62496
PROD_CHUNK = 10416
PROD_NCHUNK = 6
PROD_TAIL_OFF = NS * PROD_PER_S
PROD_TAIL = V - PROD_TAIL_OFF


def _body(idx_hbm, tf_hbm, idf_hbm, out_hbm,
          idx_v, val_v, comb_sh, sem):
    c = lax.axis_index("c")
    s = lax.axis_index("s")
    wid = s * NC + c

    tf_c = val_v.at[pl.ds(0, PROD_CHUNK)]
    idf_c = val_v.at[pl.ds(12800, PROD_CHUNK)]
    pbase = s * PROD_PER_S

    def mul_chunk(n_vec, tf_ref, idf_ref):
        def step(i, _):
            sl = pl.ds(i * LANES, LANES)
            tf_ref[sl] = tf_ref[sl] * idf_ref[sl]
            return 0
        lax.fori_loop(0, n_vec, step, 0, unroll=8)

    for k in range(PROD_NCHUNK):
        off = pbase + k * PROD_CHUNK
        pltpu.sync_copy(tf_hbm.at[pl.ds(off, PROD_CHUNK)], tf_c)
        pltpu.sync_copy(idf_hbm.at[pl.ds(off, PROD_CHUNK)], idf_c)
        mul_chunk(PROD_CHUNK // LANES, tf_c, idf_c)
        pltpu.sync_copy(tf_c, comb_sh.at[pl.ds(off, PROD_CHUNK)])

    @pl.when(s == NS - 1)
    def _tail():
        tf_t = tf_c.at[pl.ds(0, PROD_TAIL)]
        idf_t = idf_c.at[pl.ds(0, PROD_TAIL)]
        pltpu.sync_copy(tf_hbm.at[pl.ds(PROD_TAIL_OFF, PROD_TAIL)], tf_t)
        pltpu.sync_copy(idf_hbm.at[pl.ds(PROD_TAIL_OFF, PROD_TAIL)], idf_t)
        mul_chunk(PROD_TAIL // LANES, tf_t, idf_t)
        pltpu.sync_copy(tf_t, comb_sh.at[pl.ds(PROD_TAIL_OFF, PROD_TAIL)])

    plsc.subcore_barrier()

    gbase = wid * PER_W
    pltpu.sync_copy(idx_hbm.at[pl.ds(gbase, PER_W)], idx_v)
    pltpu.async_copy(comb_sh.at[idx_v], val_v, sem).wait()
    pltpu.sync_copy(val_v, out_hbm.at[pl.ds(gbase, PER_W)])


@jax.jit
def _tfidf(idx_flat, tf_flat, idf_flat):
    mesh = plsc.VectorSubcoreMesh(core_axis_name="c", subcore_axis_name="s")
    fn = pl.kernel(
        _body,
        out_type=jax.ShapeDtypeStruct((N,), jnp.float32),
        mesh=mesh,
        scratch_types=[
            pltpu.VMEM((PER_W,), jnp.int32),
            pltpu.VMEM((PER_W,), jnp.float32),
            pltpu.VMEM_SHARED((V,), jnp.float32),
            pltpu.SemaphoreType.DMA,
        ],
    )
    return fn(idx_flat, tf_flat, idf_flat)


def kernel(indices, w_es, instance, tf_table, idf_table):
    idx_flat = indices.reshape(N).astype(jnp.int32)
    tf_flat = tf_table.reshape(V)
    idf_flat = idf_table.reshape(V)
    out = _tfidf(idx_flat, tf_flat, idf_flat)
    return out.reshape(B, L)


# speedup vs baseline: 1.7842x; 1.3576x over previous
import functools

import jax
import jax.numpy as jnp
from jax import lax
from jax.experimental import pallas as pl
from jax.experimental.pallas import tpu as pltpu
from jax.experimental.pallas import tpu_sc as plsc

B, L, V = 4096, 200, 1000000
VP = V + 64
N = B * L
NC, NS, LANES = 2, 16, 16
NW = NC * NS
PER_W = N // NW

PROD_PER_S = 62496
PROD_CHUNK = 10416
PROD_NCHUNK = 6
PROD_TAIL_OFF = NS * PROD_PER_S
PROD_TAIL = VP - PROD_TAIL_OFF


def _body(idx_hbm, tf_hbm, idf_hbm, out_hbm,
          idx_v, val_v, comb_sh, sem):
    c = lax.axis_index("c")
    s = lax.axis_index("s")
    wid = s * NC + c

    tf_c = val_v.at[pl.ds(0, PROD_CHUNK)]
    idf_c = val_v.at[pl.ds(12800, PROD_CHUNK)]
    pbase = s * PROD_PER_S

    def mul_chunk(n_vec, tf_ref, idf_ref):
        def step(i, _):
            sl = pl.ds(i * LANES, LANES)
            tf_ref[sl] = tf_ref[sl] * idf_ref[sl]
            return 0
        lax.fori_loop(0, n_vec, step, 0, unroll=8)

    for k in range(PROD_NCHUNK):
        off = pbase + k * PROD_CHUNK
        pltpu.sync_copy(tf_hbm.at[pl.ds(off, PROD_CHUNK)], tf_c)
        pltpu.sync_copy(idf_hbm.at[pl.ds(off, PROD_CHUNK)], idf_c)
        mul_chunk(PROD_CHUNK // LANES, tf_c, idf_c)
        pltpu.sync_copy(tf_c, comb_sh.at[pl.ds(off, PROD_CHUNK)])

    @pl.when(s == NS - 1)
    def _tail():
        tf_t = tf_c.at[pl.ds(0, PROD_TAIL)]
        idf_t = idf_c.at[pl.ds(0, PROD_TAIL)]
        pltpu.sync_copy(tf_hbm.at[pl.ds(PROD_TAIL_OFF, PROD_TAIL)], tf_t)
        pltpu.sync_copy(idf_hbm.at[pl.ds(PROD_TAIL_OFF, PROD_TAIL)], idf_t)
        mul_chunk(PROD_TAIL // LANES, tf_t, idf_t)
        pltpu.sync_copy(tf_t, comb_sh.at[pl.ds(PROD_TAIL_OFF, PROD_TAIL)])

    plsc.subcore_barrier()

    gbase = wid * PER_W
    pltpu.sync_copy(idx_hbm.at[pl.ds(gbase, PER_W)], idx_v)
    pltpu.async_copy(comb_sh.at[idx_v], val_v, sem).wait()
    pltpu.sync_copy(val_v, out_hbm.at[pl.ds(gbase, PER_W)])


@jax.jit
def _tfidf(idx_flat, tf_lin, idf_lin):
    mesh = plsc.VectorSubcoreMesh(core_axis_name="c", subcore_axis_name="s")
    fn = pl.kernel(
        _body,
        out_type=jax.ShapeDtypeStruct((N,), jnp.float32),
        mesh=mesh,
        scratch_types=[
            pltpu.VMEM((PER_W,), jnp.int32),
            pltpu.VMEM((PER_W,), jnp.float32),
            pltpu.VMEM_SHARED((VP,), jnp.float32),
            pltpu.SemaphoreType.DMA,
        ],
    )
    return fn(idx_flat, tf_lin, idf_lin)


def kernel(indices, w_es, instance, tf_table, idf_table):
    idx_flat = indices.reshape(N).astype(jnp.int32)
    tf_lin = jnp.pad(tf_table, ((0, 64), (0, 0))).reshape(VP)
    idf_lin = jnp.pad(idf_table, ((0, 64), (0, 0))).reshape(VP)
    out = _tfidf(idx_flat, tf_lin, idf_lin)
    return out.reshape(B, L)


# speedup vs baseline: 2.3023x; 1.2903x over previous
import functools

import jax
import jax.numpy as jnp
from jax import lax
from jax.experimental import pallas as pl
from jax.experimental.pallas import tpu as pltpu
from jax.experimental.pallas import tpu_sc as plsc

B, L, V = 4096, 200, 1000000
VT = 1 << 20
VC = 1000064
N = B * L
NC, NS, LANES = 2, 16, 16
NW = NC * NS
PER_W = N // NW

PROD_PER_S = 62496
PROD_CHUNK = 10416
PROD_NCHUNK = 6
PROD_TAIL_OFF = NS * PROD_PER_S
PROD_TAIL = VC - PROD_TAIL_OFF


def _body(idx_hbm, tf_hbm, idf_hbm, out_hbm,
          idx_v, val_v, comb_sh, sem):
    c = lax.axis_index("c")
    s = lax.axis_index("s")
    wid = s * NC + c

    tf_c = val_v.at[pl.ds(0, PROD_CHUNK)]
    idf_c = val_v.at[pl.ds(12800, PROD_CHUNK)]
    pbase = s * PROD_PER_S

    def mul_chunk(n_vec, tf_ref, idf_ref):
        def step(i, _):
            sl = pl.ds(i * LANES, LANES)
            tf_ref[sl] = tf_ref[sl] * idf_ref[sl]
            return 0
        lax.fori_loop(0, n_vec, step, 0, unroll=8)

    for k in range(PROD_NCHUNK):
        off = pbase + k * PROD_CHUNK
        pltpu.sync_copy(tf_hbm.at[pl.ds(off, PROD_CHUNK)], tf_c)
        pltpu.sync_copy(idf_hbm.at[pl.ds(off, PROD_CHUNK)], idf_c)
        mul_chunk(PROD_CHUNK // LANES, tf_c, idf_c)
        pltpu.sync_copy(tf_c, comb_sh.at[pl.ds(off, PROD_CHUNK)])

    @pl.when(s == NS - 1)
    def _tail():
        tf_t = tf_c.at[pl.ds(0, PROD_TAIL)]
        idf_t = idf_c.at[pl.ds(0, PROD_TAIL)]
        pltpu.sync_copy(tf_hbm.at[pl.ds(PROD_TAIL_OFF, PROD_TAIL)], tf_t)
        pltpu.sync_copy(idf_hbm.at[pl.ds(PROD_TAIL_OFF, PROD_TAIL)], idf_t)
        mul_chunk(PROD_TAIL // LANES, tf_t, idf_t)
        pltpu.sync_copy(tf_t, comb_sh.at[pl.ds(PROD_TAIL_OFF, PROD_TAIL)])

    plsc.subcore_barrier()

    gbase = wid * PER_W
    pltpu.sync_copy(idx_hbm.at[pl.ds(gbase, PER_W)], idx_v)
    pltpu.async_copy(comb_sh.at[idx_v], val_v, sem).wait()
    pltpu.sync_copy(val_v, out_hbm.at[pl.ds(gbase, PER_W)])


@jax.jit
def _tfidf(idx_flat, tf_lin, idf_lin):
    mesh = plsc.VectorSubcoreMesh(core_axis_name="c", subcore_axis_name="s")
    fn = pl.kernel(
        _body,
        out_type=jax.ShapeDtypeStruct((N,), jnp.float32),
        mesh=mesh,
        scratch_types=[
            pltpu.VMEM((PER_W,), jnp.int32),
            pltpu.VMEM((PER_W,), jnp.float32),
            pltpu.VMEM_SHARED((VC,), jnp.float32),
            pltpu.SemaphoreType.DMA,
        ],
    )
    return fn(idx_flat, tf_lin, idf_lin)


def kernel(indices, w_es, instance, tf_table, idf_table):
    idx_flat = indices.reshape(N).astype(jnp.int32)
    tf_lin = jnp.pad(tf_table, ((0, VT - V), (0, 0))).reshape(VT)
    idf_lin = jnp.pad(idf_table, ((0, VT - V), (0, 0))).reshape(VT)
    out = _tfidf(idx_flat, tf_lin, idf_lin)
    return out.reshape(B, L)
